# baseline (speedup 1.0000x reference)
import jax
import jax.numpy as jnp
from jax.experimental import pallas as pl

TOP_K = 8
N_EXPERTS = 64
N_GROUP = 8
GROUP_SIZE = 8
TOPK_GROUP = 4
SCALE = 2.5


def _router_block(x_ref, wt_ref, bias_ref, logits_ref, idx_ref, w_ref):
    x = x_ref[...]
    wt = wt_ref[...]
    logits = jnp.dot(x, wt, preferred_element_type=jnp.float32)
    logits_ref[...] = logits
    scores = jax.nn.sigmoid(logits)
    s4c = scores + bias_ref[...]

    bt = x.shape[0]
    lane = jax.lax.broadcasted_iota(jnp.int32, (bt, N_EXPERTS), 1)
    gid = lane // GROUP_SIZE
    neg = jnp.float32(-jnp.inf)

    def partner(v, d):
        return jnp.where((lane & d) == 0, jnp.roll(v, -d, axis=1),
                         jnp.roll(v, d, axis=1))

    p = partner(s4c, 1)
    hi = jnp.maximum(s4c, p)
    lo = jnp.minimum(s4c, p)
    for d in (2, 4):
        ph = partner(hi, d)
        plo = jnp.where(hi >= ph, lo, partner(lo, d))
        hi, lo = jnp.maximum(hi, ph), jnp.maximum(jnp.minimum(hi, ph), plo)
    gs = hi + lo

    rank = jnp.zeros((bt, N_EXPERTS), jnp.int32)
    for k in range(1, N_GROUP):
        r = jnp.roll(gs, GROUP_SIZE * k, axis=1)
        beats = (r > gs) | ((r == gs) & (gid >= k))
        rank = rank + beats.astype(jnp.int32)
    mask64 = rank < TOPK_GROUP

    masked = jnp.where(mask64, s4c, 0.0)
    idxs, ws = [], []
    cur = masked
    for _ in range(TOP_K):
        vmax = jnp.max(cur, axis=1, keepdims=True)
        sel = jnp.min(jnp.where(cur == vmax, lane, N_EXPERTS), axis=1,
                      keepdims=True)
        idxs.append(sel)
        ws.append(jnp.sum(jnp.where(lane == sel, scores, 0.0), axis=1,
                          keepdims=True))
        cur = jnp.where(lane == sel, neg, cur)
    topk_idx = jnp.concatenate(idxs, axis=1)
    topk_w = jnp.concatenate(ws, axis=1)
    denom = jnp.sum(topk_w, axis=1, keepdims=True) + 1e-20
    idx_ref[...] = topk_idx
    w_ref[...] = topk_w / denom * SCALE


@jax.jit
def kernel(hidden_states, weight, e_score_correction_bias):
    b, s, h = hidden_states.shape
    t = b * s
    hs = hidden_states.reshape(t, h).astype(jnp.float32)
    wt = weight.astype(jnp.float32).T
    bias = e_score_correction_bias.astype(jnp.float32).reshape(1, N_EXPERTS)

    bt = 256
    grid = (t // bt,)
    logits, idx, w = pl.pallas_call(
        _router_block,
        grid=grid,
        in_specs=[
            pl.BlockSpec((bt, h), lambda i: (i, 0)),
            pl.BlockSpec((h, N_EXPERTS), lambda i: (0, 0)),
            pl.BlockSpec((1, N_EXPERTS), lambda i: (0, 0)),
        ],
        out_specs=[
            pl.BlockSpec((bt, N_EXPERTS), lambda i: (i, 0)),
            pl.BlockSpec((bt, TOP_K), lambda i: (i, 0)),
            pl.BlockSpec((bt, TOP_K), lambda i: (i, 0)),
        ],
        out_shape=[
            jax.ShapeDtypeStruct((t, N_EXPERTS), jnp.float32),
            jax.ShapeDtypeStruct((t, TOP_K), jnp.int32),
            jax.ShapeDtypeStruct((t, TOP_K), jnp.float32),
        ],
    )(hs, wt, bias)
    return idx, w, logits


# speedup vs baseline: 1.6093x; 1.6093x over previous
import jax
import jax.numpy as jnp
from jax.experimental import pallas as pl

TOP_K = 8
N_EXPERTS = 64
N_GROUP = 8
GROUP_SIZE = 8
TOPK_GROUP = 4
SCALE = 2.5


def _router_block(x_ref, wt_ref, bias_ref, logits_ref, idx_ref, w_ref):
    x = x_ref[...]
    wt = wt_ref[...]
    logits = jnp.dot(x, wt, preferred_element_type=jnp.float32)
    logits_ref[...] = logits
    scores = jax.nn.sigmoid(logits)
    s4c = scores + bias_ref[...]

    bt = x.shape[0]
    lane = jax.lax.broadcasted_iota(jnp.int32, (bt, N_EXPERTS), 1)
    gid = lane // GROUP_SIZE
    neg = jnp.float32(-jnp.inf)

    def partner(v, d):
        return jnp.where((lane & d) == 0, jnp.roll(v, -d, axis=1),
                         jnp.roll(v, d, axis=1))

    p = partner(s4c, 1)
    hi = jnp.maximum(s4c, p)
    lo = jnp.minimum(s4c, p)
    for d in (2, 4):
        ph = partner(hi, d)
        plo = jnp.where(hi >= ph, lo, partner(lo, d))
        hi, lo = jnp.maximum(hi, ph), jnp.maximum(jnp.minimum(hi, ph), plo)
    gs = hi + lo

    rank = jnp.zeros((bt, N_EXPERTS), jnp.int32)
    for k in range(1, N_GROUP):
        r = jnp.roll(gs, GROUP_SIZE * k, axis=1)
        beats = (r > gs) | ((r == gs) & (gid >= k))
        rank = rank + beats.astype(jnp.int32)
    mask64 = rank < TOPK_GROUP

    masked = jnp.where(mask64, s4c, 0.0)
    idxs, ws = [], []
    cur = masked
    for _ in range(TOP_K):
        vmax = jnp.max(cur, axis=1, keepdims=True)
        sel = jnp.min(jnp.where(cur == vmax, lane, N_EXPERTS), axis=1,
                      keepdims=True)
        idxs.append(sel)
        ws.append(jnp.sum(jnp.where(lane == sel, scores, 0.0), axis=1,
                          keepdims=True))
        cur = jnp.where(lane == sel, neg, cur)
    topk_idx = jnp.concatenate(idxs, axis=1)
    topk_w = jnp.concatenate(ws, axis=1)
    denom = jnp.sum(topk_w, axis=1, keepdims=True) + 1e-20
    idx_ref[...] = topk_idx
    w_ref[...] = topk_w / denom * SCALE


@jax.jit
def kernel(hidden_states, weight, e_score_correction_bias):
    b, s, h = hidden_states.shape
    t = b * s
    hs = hidden_states.reshape(t, h).astype(jnp.float32)
    wt = weight.astype(jnp.float32).T
    bias = e_score_correction_bias.astype(jnp.float32).reshape(1, N_EXPERTS)

    bt = 2048
    grid = (t // bt,)
    logits, idx, w = pl.pallas_call(
        _router_block,
        grid=grid,
        in_specs=[
            pl.BlockSpec((bt, h), lambda i: (i, 0)),
            pl.BlockSpec((h, N_EXPERTS), lambda i: (0, 0)),
            pl.BlockSpec((1, N_EXPERTS), lambda i: (0, 0)),
        ],
        out_specs=[
            pl.BlockSpec((bt, N_EXPERTS), lambda i: (i, 0)),
            pl.BlockSpec((bt, TOP_K), lambda i: (i, 0)),
            pl.BlockSpec((bt, TOP_K), lambda i: (i, 0)),
        ],
        out_shape=[
            jax.ShapeDtypeStruct((t, N_EXPERTS), jnp.float32),
            jax.ShapeDtypeStruct((t, TOP_K), jnp.int32),
            jax.ShapeDtypeStruct((t, TOP_K), jnp.float32),
        ],
    )(hs, wt, bias)
    return idx, w, logits


# speedup vs baseline: 2.1733x; 1.3504x over previous
import jax
import jax.numpy as jnp
from jax.experimental import pallas as pl

TOP_K = 8
N_EXPERTS = 64
N_GROUP = 8
GROUP_SIZE = 8
TOPK_GROUP = 4
SCALE = 2.5


def _router_block(x_ref, wt_ref, bias_ref, logits_ref, idx_ref, w_ref):
    x = x_ref[...]
    wt = wt_ref[...]
    logits = jnp.dot(x, wt, preferred_element_type=jnp.float32)
    logits_ref[...] = logits
    scores = jax.nn.sigmoid(logits)
    s4c = scores + bias_ref[...]

    bt = x.shape[0]
    lane = jax.lax.broadcasted_iota(jnp.int32, (bt, N_EXPERTS), 1)
    lanef = lane.astype(jnp.float32)
    gid = lane // GROUP_SIZE
    neg = jnp.float32(-jnp.inf)
    one = jnp.float32(1.0)

    def partner(v, d):
        return jnp.where((lane & d) == 0, jnp.roll(v, -d, axis=1),
                         jnp.roll(v, d, axis=1))

    p = partner(s4c, 1)
    hi = jnp.maximum(s4c, p)
    lo = jnp.minimum(s4c, p)
    for d in (2, 4):
        ph = partner(hi, d)
        plo = jnp.where(hi >= ph, lo, partner(lo, d))
        hi, lo = jnp.maximum(hi, ph), jnp.maximum(jnp.minimum(hi, ph), plo)
    gs = hi + lo

    rankf = jnp.zeros((bt, N_EXPERTS), jnp.float32)
    for k in (1, 2, 3, 4):
        r = jnp.roll(gs, GROUP_SIZE * k, axis=1)
        bk = jnp.where((r > gs) | ((r == gs) & (gid >= k)), one, 0.0)
        rankf = rankf + bk
        if k < 4:
            rankf = rankf + (one - jnp.roll(bk, -GROUP_SIZE * k, axis=1))
    mask64 = rankf < TOPK_GROUP

    cur = jnp.where(mask64, s4c, 0.0)
    kc = lanef * 2.0 - scores
    big = jnp.float32(200.0)
    kacc = jnp.zeros((bt, N_EXPERTS), jnp.float32)
    for j in range(TOP_K):
        vmax = jnp.max(cur, axis=1, keepdims=True)
        kmin = jnp.min(jnp.where(cur == vmax, kc, big), axis=1, keepdims=True)
        m2 = kc == kmin
        cur = jnp.where(m2, neg, cur)
        kacc = jnp.where(lane == j, kmin, kacc)
    sel_f = jnp.ceil(kacc * 0.5)
    w_all = jnp.where(lane < TOP_K, 2.0 * sel_f - kacc, 0.0)
    denom = jnp.sum(w_all, axis=1, keepdims=True) + 1e-20
    w_out = w_all * (SCALE / denom)
    idx_ref[...] = sel_f[:, :TOP_K].astype(jnp.int32)
    w_ref[...] = w_out[:, :TOP_K]


@jax.jit
def kernel(hidden_states, weight, e_score_correction_bias):
    b, s, h = hidden_states.shape
    t = b * s
    hs = hidden_states.reshape(t, h).astype(jnp.float32)
    wt = weight.astype(jnp.float32).T
    bias = e_score_correction_bias.astype(jnp.float32).reshape(1, N_EXPERTS)

    bt = 2048
    grid = (t // bt,)
    logits, idx, w = pl.pallas_call(
        _router_block,
        grid=grid,
        in_specs=[
            pl.BlockSpec((bt, h), lambda i: (i, 0)),
            pl.BlockSpec((h, N_EXPERTS), lambda i: (0, 0)),
            pl.BlockSpec((1, N_EXPERTS), lambda i: (0, 0)),
        ],
        out_specs=[
            pl.BlockSpec((bt, N_EXPERTS), lambda i: (i, 0)),
            pl.BlockSpec((bt, TOP_K), lambda i: (i, 0)),
            pl.BlockSpec((bt, TOP_K), lambda i: (i, 0)),
        ],
        out_shape=[
            jax.ShapeDtypeStruct((t, N_EXPERTS), jnp.float32),
            jax.ShapeDtypeStruct((t, TOP_K), jnp.int32),
            jax.ShapeDtypeStruct((t, TOP_K), jnp.float32),
        ],
    )(hs, wt, bias)
    return idx, w, logits
